# Initial kernel scaffold; baseline (speedup 1.0000x reference)
import jax
import jax.numpy as jnp
from jax.experimental import pallas as pl


def kernel(x):
    raise NotImplementedError("write your pallas kernel here")


# speedup vs baseline: 1.4140x; 1.4140x over previous
import jax
import jax.numpy as jnp
from jax.experimental import pallas as pl

_SIZE = 2048
_SCALE = 1.0 / (_SIZE * _SIZE)
_TR = 256
_S = _SIZE // _TR


def _wp(x2, x3, x4):
    return jnp.where(x2 == 1.0, 1000.0,
           jnp.where(x3 == 1.0, 100.0,
           jnp.where(x4 == 1.0, 10.0, 0.0))).astype(jnp.float32)


def _stencil_kernel(x2_ref, x3_ref, x4_ref, halo_ref, out_ref):
    s = pl.program_id(0)
    W = _wp(x2_ref[0], x3_ref[0], x4_ref[0])
    h = halo_ref[0]
    top = _wp(h[0:1], h[2:3], h[4:5])
    bot = _wp(h[1:2], h[3:4], h[5:6])
    wext = jnp.concatenate([top, W, bot], axis=0)
    grow = (s * _TR - 1) + jax.lax.broadcasted_iota(jnp.int32, (_TR + 2, _SIZE), 0)
    col = jax.lax.broadcasted_iota(jnp.int32, (_TR + 2, _SIZE), 1)
    valid = (grow >= 1) & (grow <= _SIZE - 2) & (col >= 1) & (col <= _SIZE - 2)
    wext = jnp.where(valid, wext, 0.0)

    wc = wext[1:-1]
    zcol = jnp.zeros((_TR, 1), jnp.float32)
    t1 = jnp.concatenate([wc[:, 1:], zcol], axis=1)
    t2 = wext[2:]
    t3 = wext[:-2]
    t4 = jnp.concatenate([zcol, wc[:, :-1]], axis=1)
    p = jnp.where(t1 != 0.0, t1,
        jnp.where(t2 != 0.0, t2,
        jnp.where(t3 != 0.0, t3,
        jnp.where(t4 != 0.0, t4, _SCALE))))
    out_ref[...] = p.T


def kernel(x):
    x = x.reshape(5, _SIZE, _SIZE)
    z = jnp.zeros((3, 1, _SIZE), x.dtype)
    tops = jnp.concatenate([z, x[2:5, _TR - 1 :: _TR, :][:, : _S - 1, :]], axis=1)
    bots = jnp.concatenate([x[2:5, _TR :: _TR, :], z], axis=1)
    halo = jnp.stack([tops[0], bots[0], tops[1], bots[1], tops[2], bots[2]],
                     axis=1)

    out = pl.pallas_call(
        _stencil_kernel,
        grid=(_S,),
        in_specs=[
            pl.BlockSpec((1, _TR, _SIZE), lambda s: (2, s, 0)),
            pl.BlockSpec((1, _TR, _SIZE), lambda s: (3, s, 0)),
            pl.BlockSpec((1, _TR, _SIZE), lambda s: (4, s, 0)),
            pl.BlockSpec((1, 6, _SIZE), lambda s: (s, 0, 0)),
        ],
        out_specs=pl.BlockSpec((_SIZE, _TR), lambda s: (0, s)),
        out_shape=jax.ShapeDtypeStruct((_SIZE, _SIZE), jnp.float32),
    )(x, x, x, halo)

    value = jnp.array([0], dtype=jnp.int32)
    return (value, out.reshape(_SIZE * _SIZE))


# speedup vs baseline: 1.7070x; 1.2072x over previous
import jax
import jax.numpy as jnp
from jax.experimental import pallas as pl
from jax.experimental.pallas import tpu as pltpu

_SIZE = 2048
_SCALE = 1.0 / (_SIZE * _SIZE)
_TR = 512
_S = _SIZE // _TR


def _wp(x2, x3, x4):
    return jnp.where(x2 == 1.0, 1000.0,
           jnp.where(x3 == 1.0, 100.0,
           jnp.where(x4 == 1.0, 10.0, 0.0))).astype(jnp.float32)


def _stencil_kernel(xany_ref, x2_ref, x3_ref, x4_ref, out_ref, halo_ref, sem):
    s = pl.program_id(0)
    i0 = s * _TR
    top_i = jnp.maximum(i0 - 1, 0)
    bot_i = jnp.minimum(i0 + _TR, _SIZE - 1)
    copies = []
    for c in range(3):
        for k, idx in enumerate((top_i, bot_i)):
            cp = pltpu.make_async_copy(
                xany_ref.at[2 + c, pl.ds(idx, 1), :],
                halo_ref.at[c, pl.ds(k, 1), :],
                sem,
            )
            cp.start()
            copies.append(cp)

    W = _wp(x2_ref[0], x3_ref[0], x4_ref[0])
    for cp in copies:
        cp.wait()
    top = _wp(halo_ref[0, 0:1], halo_ref[1, 0:1], halo_ref[2, 0:1])
    bot = _wp(halo_ref[0, 1:2], halo_ref[1, 1:2], halo_ref[2, 1:2])
    wext = jnp.concatenate([top, W, bot], axis=0)
    grow = (i0 - 1) + jax.lax.broadcasted_iota(jnp.int32, (_TR + 2, _SIZE), 0)
    col = jax.lax.broadcasted_iota(jnp.int32, (_TR + 2, _SIZE), 1)
    valid = (grow >= 1) & (grow <= _SIZE - 2) & (col >= 1) & (col <= _SIZE - 2)
    wext = jnp.where(valid, wext, 0.0)

    wc = wext[1:-1]
    zcol = jnp.zeros((_TR, 1), jnp.float32)
    t1 = jnp.concatenate([wc[:, 1:], zcol], axis=1)
    t2 = wext[2:]
    t3 = wext[:-2]
    t4 = jnp.concatenate([zcol, wc[:, :-1]], axis=1)
    p = jnp.where(t1 != 0.0, t1,
        jnp.where(t2 != 0.0, t2,
        jnp.where(t3 != 0.0, t3,
        jnp.where(t4 != 0.0, t4, _SCALE))))
    out_ref[...] = p.T


def kernel(x):
    x = x.reshape(5, _SIZE, _SIZE)
    out = pl.pallas_call(
        _stencil_kernel,
        grid=(_S,),
        in_specs=[
            pl.BlockSpec(memory_space=pl.ANY),
            pl.BlockSpec((1, _TR, _SIZE), lambda s: (2, s, 0)),
            pl.BlockSpec((1, _TR, _SIZE), lambda s: (3, s, 0)),
            pl.BlockSpec((1, _TR, _SIZE), lambda s: (4, s, 0)),
        ],
        out_specs=pl.BlockSpec((_SIZE, _TR), lambda s: (0, s)),
        out_shape=jax.ShapeDtypeStruct((_SIZE, _SIZE), jnp.float32),
        scratch_shapes=[
            pltpu.VMEM((3, 2, _SIZE), jnp.float32),
            pltpu.SemaphoreType.DMA,
        ],
    )(x, x, x, x)

    value = jnp.array([0], dtype=jnp.int32)
    return (value, out.reshape(_SIZE * _SIZE))


# speedup vs baseline: 3.2592x; 1.9093x over previous
import jax
import jax.numpy as jnp
from jax.experimental import pallas as pl
from jax.experimental.pallas import tpu as pltpu

_SIZE = 2048
_SCALE = 1.0 / (_SIZE * _SIZE)
_TA = 512
_S = _SIZE // _TA


def _wp(x2, x3, x4):
    return jnp.where(x2 == 1.0, 1000.0,
           jnp.where(x3 == 1.0, 100.0,
           jnp.where(x4 == 1.0, 10.0, 0.0))).astype(jnp.float32)


def _stencil_kernel(x2_ref, x3_ref, x4_ref, out_ref, wprev_ref, lcol_ref):
    s = pl.program_id(0)
    b = jnp.minimum(s, _S - 1)
    wcur = _wp(x2_ref[0], x3_ref[0], x4_ref[0])
    row = jax.lax.broadcasted_iota(jnp.int32, (_SIZE, _TA), 0)
    gcol = b * _TA + jax.lax.broadcasted_iota(jnp.int32, (_SIZE, _TA), 1)
    valid = (row >= 1) & (row <= _SIZE - 2) & (gcol >= 1) & (gcol <= _SIZE - 2)
    wcur = jnp.where(valid, wcur, 0.0)

    @pl.when(s == 0)
    def _():
        lcol_ref[...] = jnp.zeros((_SIZE, 1), jnp.float32)
        wprev_ref[...] = wcur

    @pl.when(s > 0)
    def _():
        ncol = jnp.where(s == _S, 0.0, wcur[:, 0:1])
        wc = wprev_ref[...]
        t1 = jnp.concatenate([wc[:, 1:], ncol], axis=1)
        zrow = jnp.zeros((1, _TA), jnp.float32)
        t2 = jnp.concatenate([wc[1:], zrow], axis=0)
        t3 = jnp.concatenate([zrow, wc[:-1]], axis=0)
        t4 = jnp.concatenate([lcol_ref[...], wc[:, :-1]], axis=1)
        p = jnp.where(t1 != 0.0, t1,
            jnp.where(t2 != 0.0, t2,
            jnp.where(t3 != 0.0, t3,
            jnp.where(t4 != 0.0, t4, _SCALE))))
        out_ref[...] = p.T.reshape(_TA, 16, 128)
        lcol_ref[...] = wprev_ref[:, _TA - 1:_TA]
        wprev_ref[...] = wcur


def kernel(x):
    x = x.reshape(5, _SIZE, _SIZE)
    out = pl.pallas_call(
        _stencil_kernel,
        grid=(_S + 1,),
        in_specs=[
            pl.BlockSpec((1, _SIZE, _TA), lambda s: (2, 0, jnp.minimum(s, _S - 1))),
            pl.BlockSpec((1, _SIZE, _TA), lambda s: (3, 0, jnp.minimum(s, _S - 1))),
            pl.BlockSpec((1, _SIZE, _TA), lambda s: (4, 0, jnp.minimum(s, _S - 1))),
        ],
        out_specs=pl.BlockSpec((_TA, 16, 128), lambda s: (jnp.maximum(s - 1, 0), 0, 0)),
        out_shape=jax.ShapeDtypeStruct((_SIZE, 16, 128), jnp.float32),
        scratch_shapes=[
            pltpu.VMEM((_SIZE, _TA), jnp.float32),
            pltpu.VMEM((_SIZE, 1), jnp.float32),
        ],
    )(x, x, x)

    value = jnp.array([0], dtype=jnp.int32)
    return (value, out.reshape(_SIZE * _SIZE))
